# baseline (speedup 1.0000x reference)
import functools

import jax
import jax.numpy as jnp
from jax import lax
from jax.experimental import pallas as pl
from jax.experimental.pallas import tpu as pltpu
from jax.experimental.pallas import tpu_sc as plsc

NC = 2
NS = 16
L = 16


@functools.lru_cache(maxsize=None)
def _build_sc_kernel(B, F, V, D):
    NW = NC * NS
    nrows = B // NW
    npass = 2
    R = nrows // npass
    ng = R * F
    nblk = R // L
    CHUNK = 128
    cpf = R // CHUNK
    NWB = F + D + 1

    mesh = plsc.VectorSubcoreMesh(core_axis_name="c", subcore_axis_name="s")

    @functools.partial(
        pl.kernel,
        mesh=mesh,
        out_type=jax.ShapeDtypeStruct((B,), jnp.float32),
        scratch_types=[
            pltpu.VMEM((ng,), jnp.int32),
            pltpu.VMEM((ng,), jnp.float32),
            pltpu.VMEM((D * R,), jnp.float32),
            pltpu.VMEM((NWB * L,), jnp.float32),
            pltpu.VMEM((R,), jnp.float32),
            pltpu.SemaphoreType.DMA,
        ],
    )
    def sc_kernel(idxt_hbm, dnst_hbm, tab_hbm, wb_hbm, out_hbm,
                  fidx_v, g_v, dense_v, wb_v, out_v, sem):
        wid = lax.axis_index("s") * NC + lax.axis_index("c")
        pltpu.sync_copy(wb_hbm, wb_v)

        def one_pass(p, carry):
            row0 = wid * nrows + p * R

            q = wid * npass + p
            pltpu.async_copy(idxt_hbm.at[pl.ds(q * ng, ng)], fidx_v, sem)
            pltpu.async_copy(dnst_hbm.at[pl.ds(q * (D * R), D * R)],
                             dense_v, sem)
            pltpu.make_async_copy(idxt_hbm.at[pl.ds(q * ng, ng)],
                                  fidx_v, sem).wait()
            pltpu.make_async_copy(dnst_hbm.at[pl.ds(q * (D * R), D * R)],
                                  dense_v, sem).wait()

            for f in range(F):
                def fire(t, cc, f=f):
                    sl = pl.ds(f * R + t * L, L)
                    idxv = fidx_v[sl] + jnp.int32(f * V)
                    pltpu.async_copy(tab_hbm.at[idxv], g_v.at[sl], sem)
                    return cc

                lax.fori_loop(0, nblk, fire, 0)
            for f in range(F):
                def drain(t, cc, f=f):
                    sl = pl.ds(f * R + t * L, L)
                    idxv = fidx_v[sl] + jnp.int32(f * V)
                    pltpu.make_async_copy(
                        tab_hbm.at[idxv], g_v.at[sl], sem).wait()
                    return cc

                lax.fori_loop(0, nblk, drain, 0)

            def blk(t, cc):
                acc = wb_v[pl.ds((F + D) * L, L)]
                for f in range(F):
                    acc = acc + (g_v[pl.ds(f * R + t * L, L)]
                                 * wb_v[pl.ds(f * L, L)])
                for d in range(D):
                    acc = acc + (dense_v[pl.ds(d * R + t * L, L)]
                                 * wb_v[pl.ds((F + d) * L, L)])
                out_v[pl.ds(t * L, L)] = acc
                return cc

            lax.fori_loop(0, nblk, blk, 0)
            pltpu.sync_copy(out_v, out_hbm.at[pl.ds(row0, R)])
            return carry

        lax.fori_loop(0, npass, one_pass, 0)

    return sc_kernel


def kernel(sparse_idx, dense, emb_tables, fc_w, fc_b):
    B, F = sparse_idx.shape
    V = emb_tables.shape[1]
    D = dense.shape[1]
    tab = emb_tables.reshape(F * V)
    wb = jnp.concatenate([fc_w.reshape(F + D), fc_b]).astype(jnp.float32)
    wb_splat = jnp.repeat(wb, L)
    NW = NC * NS
    npass = 2
    R = B // NW // npass
    idx_blocked = (sparse_idx.T.reshape(F, NW * npass, R)
                   .transpose(1, 0, 2).reshape(B * F))
    dense_blocked = (dense.T.reshape(D, NW * npass, R)
                     .transpose(1, 0, 2).reshape(B * D))
    out = _build_sc_kernel(B, F, V, D)(
        idx_blocked, dense_blocked, tab, wb_splat)
    return out.reshape(B, 1)


# speedup vs baseline: 1.0079x; 1.0079x over previous
import functools

import jax
import jax.numpy as jnp
from jax import lax
from jax.experimental import pallas as pl
from jax.experimental.pallas import tpu as pltpu
from jax.experimental.pallas import tpu_sc as plsc

NC = 2
NS = 16
L = 16


@functools.lru_cache(maxsize=None)
def _build_sc_kernel(B, F, V, D):
    NW = NC * NS
    nrows = B // NW
    npass = 2
    R = nrows // npass
    ng = R * F
    nblk = R // L
    CHUNK = 128
    cpf = R // CHUNK
    NWB = F + D + 1

    mesh = plsc.VectorSubcoreMesh(core_axis_name="c", subcore_axis_name="s")

    @functools.partial(
        pl.kernel,
        mesh=mesh,
        out_type=jax.ShapeDtypeStruct((B,), jnp.float32),
        scratch_types=[
            pltpu.VMEM((ng,), jnp.int32),
            pltpu.VMEM((ng,), jnp.float32),
            pltpu.VMEM((D * R,), jnp.float32),
            pltpu.VMEM((NWB * L,), jnp.float32),
            pltpu.VMEM((R,), jnp.float32),
            pltpu.SemaphoreType.DMA,
        ],
    )
    def sc_kernel(idxt_hbm, dnst_hbm, tab_hbm, wb_hbm, out_hbm,
                  fidx_v, g_v, dense_v, wb_v, out_v, sem):
        wid = lax.axis_index("s") * NC + lax.axis_index("c")
        pltpu.sync_copy(wb_hbm, wb_v)

        def one_pass(p, carry):
            row0 = wid * nrows + p * R

            q = wid * npass + p
            pltpu.async_copy(idxt_hbm.at[pl.ds(q * ng, ng)], fidx_v, sem)
            pltpu.async_copy(dnst_hbm.at[pl.ds(q * (D * R), D * R)],
                             dense_v, sem)
            pltpu.make_async_copy(idxt_hbm.at[pl.ds(q * ng, ng)],
                                  fidx_v, sem).wait()
            pltpu.make_async_copy(dnst_hbm.at[pl.ds(q * (D * R), D * R)],
                                  dense_v, sem).wait()

            def blk(t, cc):
                acc = wb_v[pl.ds((F + D) * L, L)]
                for f in range(F):
                    acc = acc + (g_v[pl.ds(f * R + t * L, L)]
                                 * wb_v[pl.ds(f * L, L)])
                for d in range(D):
                    acc = acc + (dense_v[pl.ds(d * R + t * L, L)]
                                 * wb_v[pl.ds((F + d) * L, L)])
                out_v[pl.ds(t * L, L)] = acc
                return cc

            lax.fori_loop(0, nblk, blk, 0)
            pltpu.sync_copy(out_v, out_hbm.at[pl.ds(row0, R)])
            return carry

        lax.fori_loop(0, npass, one_pass, 0)

    return sc_kernel


def kernel(sparse_idx, dense, emb_tables, fc_w, fc_b):
    B, F = sparse_idx.shape
    V = emb_tables.shape[1]
    D = dense.shape[1]
    tab = emb_tables.reshape(F * V)
    wb = jnp.concatenate([fc_w.reshape(F + D), fc_b]).astype(jnp.float32)
    wb_splat = jnp.repeat(wb, L)
    NW = NC * NS
    npass = 2
    R = B // NW // npass
    idx_blocked = (sparse_idx.T.reshape(F, NW * npass, R)
                   .transpose(1, 0, 2).reshape(B * F))
    dense_blocked = (dense.T.reshape(D, NW * npass, R)
                     .transpose(1, 0, 2).reshape(B * D))
    out = _build_sc_kernel(B, F, V, D)(
        idx_blocked, dense_blocked, tab, wb_splat)
    return out.reshape(B, 1)


# speedup vs baseline: 1.7840x; 1.7700x over previous
import functools

import jax
import jax.numpy as jnp
from jax import lax
from jax.experimental import pallas as pl
from jax.experimental.pallas import tpu as pltpu
from jax.experimental.pallas import tpu_sc as plsc

NC = 2
NS = 16
L = 16


@functools.lru_cache(maxsize=None)
def _build_sc_kernel(B, F, V, D):
    NW = NC * NS
    nrows = B // NW
    npass = 2
    R = nrows // npass
    ng = R * F
    nblk = R // L
    CHUNK = 128
    cpf = R // CHUNK
    NWB = F + D + 1

    mesh = plsc.VectorSubcoreMesh(core_axis_name="c", subcore_axis_name="s")

    @functools.partial(
        pl.kernel,
        mesh=mesh,
        out_type=jax.ShapeDtypeStruct((B,), jnp.float32),
        scratch_types=[
            pltpu.VMEM((ng,), jnp.int32),
            pltpu.VMEM((ng,), jnp.float32),
            pltpu.VMEM((D * R,), jnp.float32),
            pltpu.VMEM((NWB * L,), jnp.float32),
            pltpu.VMEM((R,), jnp.float32),
            pltpu.SemaphoreType.DMA,
        ],
    )
    def sc_kernel(idxt_hbm, dnst_hbm, tab_hbm, wb_hbm, out_hbm,
                  fidx_v, g_v, dense_v, wb_v, out_v, sem):
        wid = lax.axis_index("s") * NC + lax.axis_index("c")
        pltpu.sync_copy(wb_hbm, wb_v)

        def one_pass(p, carry):
            row0 = wid * nrows + p * R

            q = wid * npass + p
            pltpu.async_copy(idxt_hbm.at[pl.ds(q * ng, ng)], fidx_v, sem)
            pltpu.async_copy(dnst_hbm.at[pl.ds(q * (D * R), D * R)],
                             dense_v, sem)
            pltpu.make_async_copy(idxt_hbm.at[pl.ds(q * ng, ng)],
                                  fidx_v, sem).wait()
            pltpu.make_async_copy(dnst_hbm.at[pl.ds(q * (D * R), D * R)],
                                  dense_v, sem).wait()

            for f in range(F):
                def fire(t, cc, f=f):
                    sl = pl.ds(f * R + t * L, L)
                    idxv = fidx_v[sl] + jnp.int32(f * V)
                    pltpu.async_copy(tab_hbm.at[idxv], g_v.at[sl], sem)
                    return cc

                lax.fori_loop(0, nblk, fire, 0)
            for f in range(F):
                def drain(t, cc, f=f):
                    sl = pl.ds(f * R + t * L, L)
                    idxv = fidx_v[sl] + jnp.int32(f * V)
                    pltpu.make_async_copy(
                        tab_hbm.at[idxv], g_v.at[sl], sem).wait()
                    return cc

                lax.fori_loop(0, nblk, drain, 0)

            def blk(t, cc):
                acc = wb_v[pl.ds((F + D) * L, L)]
                for f in range(F):
                    acc = acc + (g_v[pl.ds(f * R + t * L, L)]
                                 * wb_v[pl.ds(f * L, L)])
                for d in range(D):
                    acc = acc + (dense_v[pl.ds(d * R + t * L, L)]
                                 * wb_v[pl.ds((F + d) * L, L)])
                out_v[pl.ds(t * L, L)] = acc
                return cc

            lax.fori_loop(0, nblk, blk, 0)
            pltpu.sync_copy(out_v, out_hbm.at[pl.ds(row0, R)])
            return carry

        lax.fori_loop(0, npass, one_pass, 0)

    return sc_kernel


def kernel(sparse_idx, dense, emb_tables, fc_w, fc_b):
    B, F = sparse_idx.shape
    V = emb_tables.shape[1]
    D = dense.shape[1]
    tab = jnp.concatenate([emb_tables[f, :, 0] for f in range(F)])
    wb = jnp.concatenate([fc_w.reshape(F + D), fc_b]).astype(jnp.float32)
    wb_splat = jnp.repeat(wb, L)
    NW = NC * NS
    npass = 2
    R = B // NW // npass
    idx_blocked = (sparse_idx.T.reshape(F, NW * npass, R)
                   .transpose(1, 0, 2).reshape(B * F))
    dense_blocked = (dense.T.reshape(D, NW * npass, R)
                     .transpose(1, 0, 2).reshape(B * D))
    out = _build_sc_kernel(B, F, V, D)(
        idx_blocked, dense_blocked, tab, wb_splat)
    return out.reshape(B, 1)
